# baseline (speedup 1.0000x reference)
import functools

import jax
import jax.numpy as jnp
from jax import lax
from jax.experimental import pallas as pl
from jax.experimental.pallas import tpu as pltpu
from jax.experimental.pallas import tpu_sc as plsc

B = 16
N = 2048
M = 2048
L = 16
NC = 2
NS = 16
NW = NC * NS
ROWS = N // 2
RB = 8
INF = float("inf")


def _chamfer_body(gt_hbm, pred_hbm, rowmin_hbm, colmin_hbm,
                  gt_v, pred_v, colmin_v, rowmin_v, acc_tile):
    wid = lax.axis_index("s") * NC + lax.axis_index("c")
    b = wid // 2
    h = wid % 2

    pltpu.sync_copy(gt_hbm.at[b], gt_v)
    pltpu.sync_copy(pred_hbm.at[b], pred_v)

    inf16 = jnp.full((L,), INF, jnp.float32)

    def init_col(jc, _):
        colmin_v[pl.ds(jc * L, L)] = inf16
        return 0

    lax.fori_loop(0, M // L, init_col, 0)

    row0 = h * ROWS
    lane = lax.iota(jnp.int32, L)

    def row_block(rb, _):
        base = row0 + rb * L
        gx16 = gt_v[0, pl.ds(base, L)]
        gy16 = gt_v[1, pl.ds(base, L)]
        gz16 = gt_v[2, pl.ds(base, L)]
        for sub in range(L // RB):
            g = []
            for r in range(RB):
                ln = sub * RB + r
                g.append((jnp.full((L,), gx16[ln]),
                          jnp.full((L,), gy16[ln]),
                          jnp.full((L,), gz16[ln])))

            def col_chunk(jc, accs):
                px = pred_v[0, pl.ds(jc * L, L)]
                py = pred_v[1, pl.ds(jc * L, L)]
                pz = pred_v[2, pl.ds(jc * L, L)]
                cm = colmin_v[pl.ds(jc * L, L)]
                out = []
                for r in range(RB):
                    dx = px - g[r][0]
                    dy = py - g[r][1]
                    dz = pz - g[r][2]
                    d = dx * dx + dy * dy + dz * dz
                    out.append(jnp.minimum(accs[r], d))
                    cm = jnp.minimum(cm, d)
                colmin_v[pl.ds(jc * L, L)] = cm
                return tuple(out)

            accs = lax.fori_loop(0, M // L, col_chunk, (inf16,) * RB)
            for r in range(RB):
                acc_tile[sub * RB + r, :] = accs[r]
        cols = [plsc.load_gather(acc_tile,
                                 [lane, jnp.full((L,), c, jnp.int32)])
                for c in range(L)]
        while len(cols) > 1:
            cols = [jnp.minimum(cols[2 * i], cols[2 * i + 1])
                    for i in range(len(cols) // 2)]
        rowmin_v[pl.ds(rb * L, L)] = cols[0]
        return 0

    lax.fori_loop(0, ROWS // L, row_block, 0)

    pltpu.sync_copy(rowmin_v, rowmin_hbm.at[wid])
    pltpu.sync_copy(colmin_v, colmin_hbm.at[wid])


@jax.jit
def kernel(gt, pred):
    gt_t = jnp.transpose(gt, (0, 2, 1))
    pred_t = jnp.transpose(pred, (0, 2, 1))

    mesh = plsc.VectorSubcoreMesh(core_axis_name="c", subcore_axis_name="s")
    rowmin, colmin = pl.kernel(
        _chamfer_body,
        out_type=(
            jax.ShapeDtypeStruct((NW, ROWS), jnp.float32),
            jax.ShapeDtypeStruct((NW, M), jnp.float32),
        ),
        mesh=mesh,
        compiler_params=pltpu.CompilerParams(needs_layout_passes=False),
        scratch_types=[
            pltpu.VMEM((3, N), jnp.float32),
            pltpu.VMEM((3, M), jnp.float32),
            pltpu.VMEM((M,), jnp.float32),
            pltpu.VMEM((ROWS,), jnp.float32),
            pltpu.VMEM((L, L), jnp.float32),
        ],
    )(gt_t, pred_t)

    cost_for = rowmin.reshape(B, N)
    cost_bac = colmin.reshape(B, 2, M).min(axis=1)
    return jnp.mean(cost_for) + jnp.mean(cost_bac)
